# baseline (speedup 1.0000x reference)
import functools

import jax
import jax.numpy as jnp
from jax import lax
from jax.experimental import pallas as pl
from jax.experimental.pallas import tpu as pltpu
from jax.experimental.pallas import tpu_sc as plsc

VOCAB = 100000
D = 64
B = 4096
CTX = 20

NW = 32
BPW = B // NW
CHUNKS = 4
ROWS_PER_CHUNK = BPW // CHUNKS
IDX_PER_CHUNK = ROWS_PER_CHUNK * CTX
GATHER = 128
SUB = IDX_PER_CHUNK // GATHER
LANES = 16

VT = 1024


def _mean_pool_sc(idx3, emb):
    mesh = plsc.VectorSubcoreMesh(core_axis_name="c", subcore_axis_name="s")

    @functools.partial(
        pl.kernel,
        mesh=mesh,
        compiler_params=pltpu.CompilerParams(use_tc_tiling_on_sc=False),
        out_type=jax.ShapeDtypeStruct((NW, BPW, D), jnp.float32),
        scratch_types=[
            pltpu.VMEM((CTX, GATHER), jnp.int32),
            pltpu.VMEM((IDX_PER_CHUNK, D), jnp.float32),
            pltpu.VMEM((BPW, D), jnp.float32),
            pltpu.SemaphoreType.DMA,
        ],
    )
    def k(idx_hbm, emb_hbm, out_hbm, idx_v, rows_v, acc_v, sem):
        wid = lax.axis_index("s") * 2 + lax.axis_index("c")
        pltpu.sync_copy(idx_hbm.at[wid], idx_v)
        for c in range(CHUNKS):
            handles = [
                pltpu.async_copy(
                    emb_hbm.at[idx_v.at[c * SUB + j]],
                    rows_v.at[pl.ds(j * GATHER, GATHER)],
                    sem,
                )
                for j in range(SUB)
            ]
            for h in handles:
                h.wait()

            def body(r, carry, c=c):
                base = r * CTX
                for g in range(D // LANES):
                    s = rows_v[base, pl.ds(g * LANES, LANES)]
                    for kk in range(1, CTX):
                        s = s + rows_v[base + kk, pl.ds(g * LANES, LANES)]
                    acc_v[c * ROWS_PER_CHUNK + r, pl.ds(g * LANES, LANES)] = (
                        s * (1.0 / CTX)
                    )
                return carry

            lax.fori_loop(0, ROWS_PER_CHUNK, body, 0)
        pltpu.sync_copy(acc_v, out_hbm.at[wid])

    return k(idx3, emb)


def _logits_tc(mean, w, b2):
    def body(mean_ref, w_ref, b_ref, out_ref, mbf_ref):
        @pl.when(pl.program_id(0) == 0)
        def _():
            mbf_ref[...] = mean_ref[...].astype(jnp.bfloat16)

        wt = w_ref[...].astype(jnp.bfloat16)
        acc = lax.dot_general(
            mbf_ref[...], wt, (((1,), (1,)), ((), ())),
            preferred_element_type=jnp.float32,
        )
        out_ref[...] = acc + b_ref[...]

    return pl.pallas_call(
        body,
        grid=(pl.cdiv(VOCAB, VT),),
        in_specs=[
            pl.BlockSpec((B, D), lambda j: (0, 0)),
            pl.BlockSpec((VT, D), lambda j: (j, 0)),
            pl.BlockSpec((1, VT), lambda j: (0, j)),
        ],
        out_specs=pl.BlockSpec((B, VT), lambda j: (0, j)),
        out_shape=jax.ShapeDtypeStruct((B, VOCAB), jnp.float32),
        scratch_shapes=[pltpu.VMEM((B, D), jnp.bfloat16)],
    )(mean, w, b2)


def kernel(context_indices, embeddings, linear_w, linear_b):
    idx3 = context_indices.astype(jnp.int32).reshape(NW, CTX, GATHER)
    mean = _mean_pool_sc(idx3, embeddings).reshape(B, D)
    return _logits_tc(mean, linear_w, linear_b.reshape(1, VOCAB))


# speedup vs baseline: 3.3499x; 3.3499x over previous
import functools

import jax
import jax.numpy as jnp
from jax import lax
from jax.experimental import pallas as pl
from jax.experimental.pallas import tpu as pltpu
from jax.experimental.pallas import tpu_sc as plsc

VOCAB = 100000
D = 64
B = 4096
CTX = 20

NW = 32
BPW = B // NW
CHUNKS = 4
ROWS_PER_CHUNK = BPW // CHUNKS
IDX_PER_CHUNK = ROWS_PER_CHUNK * CTX
GATHER = 128
SUB = IDX_PER_CHUNK // GATHER
LANES = 16

VT = 1024


def _mean_pool_sc(idx3, emb):
    mesh = plsc.VectorSubcoreMesh(core_axis_name="c", subcore_axis_name="s")

    @functools.partial(
        pl.kernel,
        mesh=mesh,
        compiler_params=pltpu.CompilerParams(use_tc_tiling_on_sc=False),
        out_type=jax.ShapeDtypeStruct((NW, BPW, D), jnp.float32),
        scratch_types=[
            pltpu.VMEM((CTX, GATHER), jnp.int32),
            pltpu.VMEM((IDX_PER_CHUNK, D), jnp.float32),
            pltpu.VMEM((BPW, D), jnp.float32),
            pltpu.SemaphoreType.DMA,
        ],
    )
    def k(idx_hbm, emb_hbm, out_hbm, idx_v, rows_v, acc_v, sem):
        wid = lax.axis_index("s") * 2 + lax.axis_index("c")
        pltpu.sync_copy(idx_hbm.at[wid], idx_v)
        for c in range(CHUNKS):
            handles = [
                pltpu.async_copy(
                    emb_hbm.at[idx_v.at[c * SUB + j]],
                    rows_v.at[pl.ds(j * GATHER, GATHER)],
                    sem,
                )
                for j in range(SUB)
            ]
            for h in handles:
                h.wait()

            def body(r, carry, c=c):
                base = r * CTX
                for g in range(D // LANES):
                    s = rows_v[base, pl.ds(g * LANES, LANES)]
                    for kk in range(1, CTX):
                        s = s + rows_v[base + kk, pl.ds(g * LANES, LANES)]
                    acc_v[c * ROWS_PER_CHUNK + r, pl.ds(g * LANES, LANES)] = (
                        s * (1.0 / CTX)
                    )
                return carry

            lax.fori_loop(0, ROWS_PER_CHUNK, body, 0)
        pltpu.sync_copy(acc_v, out_hbm.at[wid])

    return k(idx3, emb)


def _logits_tc(wt_aug, mean_aug):
    def body(w_ref, mean_ref, out_ref):
        out_ref[...] = lax.dot_general(
            w_ref[...], mean_ref[...], (((0,), (1,)), ((), ())),
            preferred_element_type=jnp.float32,
        )

    return pl.pallas_call(
        body,
        grid=(pl.cdiv(VOCAB, VT),),
        in_specs=[
            pl.BlockSpec((D + 1, VT), lambda j: (0, j)),
            pl.BlockSpec((B, D + 1), lambda j: (0, 0)),
        ],
        out_specs=pl.BlockSpec((VT, B), lambda j: (j, 0)),
        out_shape=jax.ShapeDtypeStruct((VOCAB, B), jnp.float32),
    )(wt_aug, mean_aug)


def kernel(context_indices, embeddings, linear_w, linear_b):
    idx3 = context_indices.astype(jnp.int32).reshape(NW, CTX, GATHER)
    mean = _mean_pool_sc(idx3, embeddings).reshape(B, D)
    wt_aug = jnp.concatenate(
        [linear_w.T, linear_b[None, :]], axis=0).astype(jnp.bfloat16)
    mean_aug = jnp.concatenate(
        [mean, jnp.ones((B, 1), jnp.float32)], axis=1).astype(jnp.bfloat16)
    return _logits_tc(wt_aug, mean_aug).T


# speedup vs baseline: 3.3907x; 1.0122x over previous
import functools

import jax
import jax.numpy as jnp
from jax import lax
from jax.experimental import pallas as pl
from jax.experimental.pallas import tpu as pltpu
from jax.experimental.pallas import tpu_sc as plsc

VOCAB = 100000
D = 64
B = 4096
CTX = 20

NW = 32
BPW = B // NW
CHUNKS = 4
ROWS_PER_CHUNK = BPW // CHUNKS
IDX_PER_CHUNK = ROWS_PER_CHUNK * CTX
GATHER = 128
SUB = IDX_PER_CHUNK // GATHER
LANES = 16

VT = 1024


def _mean_pool_sc(idx3, emb):
    mesh = plsc.VectorSubcoreMesh(core_axis_name="c", subcore_axis_name="s")

    @functools.partial(
        pl.kernel,
        mesh=mesh,
        compiler_params=pltpu.CompilerParams(use_tc_tiling_on_sc=False),
        out_type=jax.ShapeDtypeStruct((NW, BPW, D), jnp.float32),
        scratch_types=[
            pltpu.VMEM((CTX, GATHER), jnp.int32),
            pltpu.VMEM((2, IDX_PER_CHUNK, D), jnp.float32),
            pltpu.VMEM((BPW, D), jnp.float32),
            pltpu.SemaphoreType.DMA,
            pltpu.SemaphoreType.DMA,
        ],
    )
    def k(idx_hbm, emb_hbm, out_hbm, idx_v, rows_v, acc_v, sem0, sem1):
        wid = lax.axis_index("s") * 2 + lax.axis_index("c")
        sems = (sem0, sem1)
        pltpu.sync_copy(idx_hbm.at[wid], idx_v)

        def fire(c):
            return [
                pltpu.async_copy(
                    emb_hbm.at[idx_v.at[c * SUB + j]],
                    rows_v.at[c % 2, pl.ds(j * GATHER, GATHER)],
                    sems[c % 2],
                )
                for j in range(SUB)
            ]

        pending = fire(0)
        for c in range(CHUNKS):
            for h in pending:
                h.wait()
            if c + 1 < CHUNKS:
                pending = fire(c + 1)

            def body(r, carry, c=c):
                base = r * CTX
                buf = c % 2
                for g in range(D // LANES):
                    s = rows_v[buf, base, pl.ds(g * LANES, LANES)]
                    for kk in range(1, CTX):
                        s = s + rows_v[buf, base + kk, pl.ds(g * LANES, LANES)]
                    acc_v[c * ROWS_PER_CHUNK + r, pl.ds(g * LANES, LANES)] = (
                        s * (1.0 / CTX)
                    )
                return carry

            lax.fori_loop(0, ROWS_PER_CHUNK, body, 0)
        pltpu.sync_copy(acc_v, out_hbm.at[wid])

    return k(idx3, emb)


def _logits_tc(wt_aug, mean_aug):
    def body(w_ref, mean_ref, out_ref):
        out_ref[...] = lax.dot_general(
            w_ref[...], mean_ref[...], (((0,), (1,)), ((), ())),
            preferred_element_type=jnp.float32,
        )

    return pl.pallas_call(
        body,
        grid=(pl.cdiv(VOCAB, VT),),
        in_specs=[
            pl.BlockSpec((D + 1, VT), lambda j: (0, j)),
            pl.BlockSpec((B, D + 1), lambda j: (0, 0)),
        ],
        out_specs=pl.BlockSpec((VT, B), lambda j: (j, 0)),
        out_shape=jax.ShapeDtypeStruct((VOCAB, B), jnp.float32),
    )(wt_aug, mean_aug)


def kernel(context_indices, embeddings, linear_w, linear_b):
    idx3 = context_indices.astype(jnp.int32).reshape(NW, CTX, GATHER)
    mean = _mean_pool_sc(idx3, embeddings).reshape(B, D)
    wt_aug = jnp.concatenate(
        [linear_w.T, linear_b[None, :]], axis=0).astype(jnp.bfloat16)
    mean_aug = jnp.concatenate(
        [mean, jnp.ones((B, 1), jnp.float32)], axis=1).astype(jnp.bfloat16)
    return _logits_tc(wt_aug, mean_aug).T
